# baseline (speedup 1.0000x reference)
import jax
import jax.numpy as jnp
from jax import lax
from jax.experimental import pallas as pl
from jax.experimental.pallas import tpu as pltpu
from jax.experimental.pallas import tpu_sc as plsc

K = 64
N = 32768
ROWS = 128
NV = N // 16
NWORKERS = 32
H1, H2, H3 = 2048, 1024, 1024

TC_ROWS = 48
TC_BLOCK = 8
CHUNKS = N // 128
LANES = 128


def _bits_of(v):
    return lax.bitcast_convert_type(v, jnp.int32) & jnp.int32(0x7FFFFFFF)


def _find_bucket(hist_ref, nbuckets, target):
    nv = nbuckets // 16

    def body_a(i, c):
        run, vi, run_before = c
        idx = nv - 1 - i
        v = hist_ref[pl.ds(idx * 16, 16)]
        tot = jnp.sum(v)
        newrun = run + tot
        hit = (newrun >= target) & (run < target)
        vi = jnp.where(hit, idx, vi)
        run_before = jnp.where(hit, run, run_before)
        return newrun, vi, run_before

    _, vi, run_before = lax.fori_loop(
        0, nv, body_a, (jnp.int32(0), jnp.int32(0), jnp.int32(0)))
    v = hist_ref[pl.ds(vi * 16, 16)]
    suf = lax.rev(plsc.cumsum(lax.rev(v, (0,))), (0,))
    mask = (run_before + suf) >= target
    jstar = jnp.sum(mask.astype(jnp.int32)) - 1
    sel = jnp.arange(16, dtype=jnp.int32) == jstar
    vj = jnp.sum(jnp.where(sel, v, 0))
    sufj = jnp.sum(jnp.where(sel, suf, 0))
    return vi * 16 + jstar, run_before + sufj - vj, vj


def _do_row(x_hbm, out_hbm, xv, h1, h2, h3, row):
    pltpu.sync_copy(x_hbm.at[row], xv)

    zeros16 = jnp.zeros((16,), jnp.int32)
    ones16 = jnp.ones((16,), jnp.int32)

    @plsc.parallel_loop(0, H1 // 16, unroll=8)
    def _(i):
        h1[pl.ds(i * 16, 16)] = zeros16

    @plsc.parallel_loop(0, H2 // 16, unroll=8)
    def _(i):
        h2[pl.ds(i * 16, 16)] = zeros16
        h3[pl.ds(i * 16, 16)] = zeros16

    @plsc.parallel_loop(0, NV, unroll=8)
    def _(i):
        bits = _bits_of(xv[pl.ds(i * 16, 16)])
        plsc.addupdate_scatter(h1, [bits >> 20], ones16)

    b1, above1, _ = _find_bucket(h1, H1, K)
    r2 = K - above1

    @plsc.parallel_loop(0, NV, unroll=8)
    def _(i):
        bits = _bits_of(xv[pl.ds(i * 16, 16)])
        plsc.addupdate_scatter(h2, [(bits >> 10) & 0x3FF], ones16,
                               mask=(bits >> 20) == b1)

    b2, above2, _ = _find_bucket(h2, H2, r2)
    r3 = r2 - above2
    hi21 = (b1 << 10) | b2

    @plsc.parallel_loop(0, NV, unroll=8)
    def _(i):
        bits = _bits_of(xv[pl.ds(i * 16, 16)])
        plsc.addupdate_scatter(h3, [bits & 0x3FF], ones16,
                               mask=(bits >> 10) == hi21)

    b3, above3, meq = _find_bucket(h3, H3, r3)
    t = (hi21 << 10) | b3
    r = r3 - above3

    def simple(_):
        @plsc.parallel_loop(0, NV, unroll=8)
        def _(i):
            v = xv[pl.ds(i * 16, 16)]
            keep = _bits_of(v) >= t
            xv[pl.ds(i * 16, 16)] = jnp.where(keep, v, 0.0)
        return 0

    def careful(_):
        def body(i, run):
            v = xv[pl.ds(i * 16, 16)]
            bits = _bits_of(v)
            eq = bits == t
            eqi = eq.astype(jnp.int32)
            excl = plsc.cumsum(eqi) - eqi
            keep = (bits > t) | (eq & ((excl + run) < r))
            xv[pl.ds(i * 16, 16)] = jnp.where(keep, v, 0.0)
            return run + jnp.sum(eqi)
        lax.fori_loop(0, NV, body, jnp.int32(0))
        return 0

    lax.cond(r == meq, simple, careful, 0)
    pltpu.sync_copy(xv, out_hbm.at[row])


def _sc_topk(x_hbm, out_hbm, xv, h1, h2, h3):
    m = ROWS - TC_ROWS
    wid = lax.axis_index("s") * 2 + lax.axis_index("c")
    nw = (m - wid + NWORKERS - 1) // NWORKERS

    def rb(j, _):
        _do_row(x_hbm, out_hbm, xv, h1, h2, h3, wid + j * NWORKERS)
        return 0

    lax.fori_loop(0, nw, rb, 0)


def _sc_call(x_tail):
    f = pl.kernel(
        _sc_topk,
        out_type=jax.ShapeDtypeStruct((ROWS - TC_ROWS, N), jnp.float32),
        mesh=plsc.VectorSubcoreMesh(core_axis_name="c", subcore_axis_name="s",
                                    num_cores=2, num_subcores=16),
        scratch_types=[
            pltpu.VMEM((N,), jnp.float32),
            pltpu.VMEM((H1,), jnp.int32),
            pltpu.VMEM((H2,), jnp.int32),
            pltpu.VMEM((H3,), jnp.int32),
        ],
        compiler_params=pltpu.CompilerParams(needs_layout_passes=False),
    )
    return f(x_tail)


def _tc_body(x_ref, o_ref):
    x = x_ref[...]
    bits = lax.bitcast_convert_type(x, jnp.int32) & jnp.int32(0x7FFFFFFF)

    def count_ge(t):
        return jnp.sum((bits >= t).astype(jnp.int32), axis=(1, 2),
                       keepdims=True)

    def body(_, carry):
        lo, hi = carry
        mid = lo + (hi - lo) // 2
        big = count_ge(mid) >= K
        return jnp.where(big, mid, lo), jnp.where(big, hi, mid)

    rr = x.shape[0]
    lo0 = jnp.zeros((rr, 1, 1), jnp.int32)
    hi0 = jnp.full((rr, 1, 1), 0x7F800000, jnp.int32)
    lo, _ = lax.fori_loop(0, 31, body, (lo0, hi0), unroll=True)
    gt = bits > lo
    eq = bits == lo
    n_gt = jnp.sum(gt.astype(jnp.int32), axis=(1, 2), keepdims=True)
    need = K - n_gt

    eqf = eq.astype(jnp.float32)
    lane_tri = (lax.broadcasted_iota(jnp.int32, (LANES, LANES), 0)
                < lax.broadcasted_iota(jnp.int32, (LANES, LANES), 1)
                ).astype(jnp.float32)
    lane_pre = lax.dot_general(eqf, lane_tri, (((2,), (0,)), ((), ())),
                               preferred_element_type=jnp.float32)
    chunk_tot = jnp.sum(eqf, axis=2)
    chunk_tri = (lax.broadcasted_iota(jnp.int32, (CHUNKS, CHUNKS), 0)
                 < lax.broadcasted_iota(jnp.int32, (CHUNKS, CHUNKS), 1)
                 ).astype(jnp.float32)
    chunk_pre = lax.dot_general(chunk_tot, chunk_tri,
                                (((1,), (0,)), ((), ())),
                                preferred_element_type=jnp.float32)
    prefix = lane_pre + chunk_pre[:, :, None]

    keep = gt | (eq & (prefix < need.astype(jnp.float32)))
    o_ref[...] = jnp.where(keep, x, 0.0)


def _tc_call(x_head):
    x3 = x_head.reshape(TC_ROWS, CHUNKS, LANES)
    out = pl.pallas_call(
        _tc_body,
        out_shape=jax.ShapeDtypeStruct(x3.shape, x3.dtype),
        grid=(TC_ROWS // TC_BLOCK,),
        in_specs=[pl.BlockSpec((TC_BLOCK, CHUNKS, LANES),
                               lambda i: (i, 0, 0))],
        out_specs=pl.BlockSpec((TC_BLOCK, CHUNKS, LANES),
                               lambda i: (i, 0, 0)),
        compiler_params=pltpu.CompilerParams(
            dimension_semantics=("arbitrary",)),
    )(x3)
    return out.reshape(TC_ROWS, N)


@jax.jit
def kernel(x):
    sc_out = _sc_call(x[TC_ROWS:])
    tc_out = _tc_call(x[:TC_ROWS])
    return jnp.concatenate([tc_out, sc_out], axis=0)


# speedup vs baseline: 1.8763x; 1.8763x over previous
import jax
import jax.numpy as jnp
from jax import lax
from jax.experimental import pallas as pl
from jax.experimental.pallas import tpu as pltpu
from jax.experimental.pallas import tpu_sc as plsc

K = 64
N = 32768
ROWS = 128
NV = N // 16
NWORKERS = 32
ROWS_PER_W = ROWS // NWORKERS
H1, H2, H3 = 2048, 1024, 1024


def _bits_of(v):
    return lax.bitcast_convert_type(v, jnp.int32) & jnp.int32(0x7FFFFFFF)


def _stage_b(v, run_before, target):
    suf = lax.rev(plsc.cumsum(lax.rev(v, (0,))), (0,))
    mask = (run_before + suf) >= target
    jstar = jnp.sum(mask.astype(jnp.int32)) - 1
    sel = jnp.arange(16, dtype=jnp.int32) == jstar
    vj = jnp.sum(jnp.where(sel, v, 0))
    sufj = jnp.sum(jnp.where(sel, suf, 0))
    return jstar, run_before + sufj - vj, vj


def _find_bucket(h_ref, tot_ref, n_tot_vregs, target):
    def body_a(i, c):
        run, gi, run_before = c
        idx = n_tot_vregs - 1 - i
        v = tot_ref[pl.ds(idx * 16, 16)]
        tot = jnp.sum(v)
        newrun = run + tot
        hit = (newrun >= target) & (run < target)
        gi = jnp.where(hit, idx, gi)
        run_before = jnp.where(hit, run, run_before)
        return newrun, gi, run_before

    _, gi, run_g = lax.fori_loop(
        0, n_tot_vregs, body_a, (jnp.int32(0), jnp.int32(0), jnp.int32(0)))
    vt = tot_ref[pl.ds(gi * 16, 16)]
    j1, above1, _ = _stage_b(vt, run_g, target)
    vi = gi * 16 + j1
    vh = h_ref[pl.ds(vi * 16, 16)]
    j2, above2, cnt = _stage_b(vh, above1, target)
    return vi * 16 + j2, above2, cnt


def _totals(h_ref, tot_ref, n_hist_vregs):
    lane = jnp.arange(16, dtype=jnp.int32)
    last = lane == 15

    @plsc.parallel_loop(0, n_hist_vregs, unroll=8)
    def _(i):
        s = plsc.cumsum(h_ref[pl.ds(i * 16, 16)])
        plsc.store_scatter(tot_ref, [lane * 0 + i], s, mask=last)


def _do_row(x_hbm, out_hbm, xv, h1, h2, h3, tot, row):
    pltpu.sync_copy(x_hbm.at[row], xv)

    zeros16 = jnp.zeros((16,), jnp.int32)
    ones16 = jnp.ones((16,), jnp.int32)

    @plsc.parallel_loop(0, H1 // 16, unroll=8)
    def _(i):
        h1[pl.ds(i * 16, 16)] = zeros16

    @plsc.parallel_loop(0, H2 // 16, unroll=8)
    def _(i):
        h2[pl.ds(i * 16, 16)] = zeros16
        h3[pl.ds(i * 16, 16)] = zeros16

    @plsc.parallel_loop(0, NV, unroll=8)
    def _(i):
        bits = _bits_of(xv[pl.ds(i * 16, 16)])
        plsc.addupdate_scatter(h1, [bits >> 20], ones16)

    _totals(h1, tot, H1 // 16)
    b1, above1, _ = _find_bucket(h1, tot, H1 // 256, K)
    r2 = K - above1

    @plsc.parallel_loop(0, NV, unroll=8)
    def _(i):
        bits = _bits_of(xv[pl.ds(i * 16, 16)])
        plsc.addupdate_scatter(h2, [(bits >> 10) & 0x3FF], ones16,
                               mask=(bits >> 20) == b1)

    _totals(h2, tot, H2 // 16)
    b2, above2, _ = _find_bucket(h2, tot, H2 // 256, r2)
    r3 = r2 - above2
    hi21 = (b1 << 10) | b2

    @plsc.parallel_loop(0, NV, unroll=8)
    def _(i):
        bits = _bits_of(xv[pl.ds(i * 16, 16)])
        plsc.addupdate_scatter(h3, [bits & 0x3FF], ones16,
                               mask=(bits >> 10) == hi21)

    _totals(h3, tot, H3 // 16)
    b3, above3, meq = _find_bucket(h3, tot, H3 // 256, r3)
    t = (hi21 << 10) | b3
    r = r3 - above3

    def simple(_):
        @plsc.parallel_loop(0, NV, unroll=8)
        def _(i):
            v = xv[pl.ds(i * 16, 16)]
            keep = _bits_of(v) >= t
            xv[pl.ds(i * 16, 16)] = jnp.where(keep, v, 0.0)
        return 0

    def careful(_):
        def body(i, run):
            v = xv[pl.ds(i * 16, 16)]
            bits = _bits_of(v)
            eq = bits == t
            eqi = eq.astype(jnp.int32)
            excl = plsc.cumsum(eqi) - eqi
            keep = (bits > t) | (eq & ((excl + run) < r))
            xv[pl.ds(i * 16, 16)] = jnp.where(keep, v, 0.0)
            return run + jnp.sum(eqi)
        lax.fori_loop(0, NV, body, jnp.int32(0))
        return 0

    lax.cond(r == meq, simple, careful, 0)
    pltpu.sync_copy(xv, out_hbm.at[row])


def _sc_topk(x_hbm, out_hbm, xv, h1, h2, h3, tot):
    wid = lax.axis_index("s") * 2 + lax.axis_index("c")

    def rb(j, _):
        _do_row(x_hbm, out_hbm, xv, h1, h2, h3, tot,
                wid * ROWS_PER_W + j)
        return 0

    lax.fori_loop(0, ROWS_PER_W, rb, 0)


@jax.jit
def kernel(x):
    f = pl.kernel(
        _sc_topk,
        out_type=jax.ShapeDtypeStruct((ROWS, N), jnp.float32),
        mesh=plsc.VectorSubcoreMesh(core_axis_name="c", subcore_axis_name="s",
                                    num_cores=2, num_subcores=16),
        scratch_types=[
            pltpu.VMEM((N,), jnp.float32),
            pltpu.VMEM((H1,), jnp.int32),
            pltpu.VMEM((H2,), jnp.int32),
            pltpu.VMEM((H3,), jnp.int32),
            pltpu.VMEM((H1 // 16,), jnp.int32),
        ],
        compiler_params=pltpu.CompilerParams(needs_layout_passes=False),
    )
    return f(x)


# speedup vs baseline: 1.9552x; 1.0421x over previous
import jax
import jax.numpy as jnp
from jax import lax
from jax.experimental import pallas as pl
from jax.experimental.pallas import tpu as pltpu
from jax.experimental.pallas import tpu_sc as plsc

K = 64
N = 32768
ROWS = 128
NV = N // 16
NWORKERS = 32
ROWS_PER_W = ROWS // NWORKERS
H1, H2, H3 = 2048, 1024, 1024


def _bits_of(v):
    return lax.bitcast_convert_type(v, jnp.int32) & jnp.int32(0x7FFFFFFF)


def _stage_b(v, run_before, target):
    suf = lax.rev(plsc.cumsum(lax.rev(v, (0,))), (0,))
    mask = (run_before + suf) >= target
    jstar = jnp.sum(mask.astype(jnp.int32)) - 1
    sel = jnp.arange(16, dtype=jnp.int32) == jstar
    vj = jnp.sum(jnp.where(sel, v, 0))
    sufj = jnp.sum(jnp.where(sel, suf, 0))
    return jstar, run_before + sufj - vj, vj


def _find_bucket(h_ref, tot_ref, n_tot_vregs, target):
    def body_a(i, c):
        run, gi, run_before = c
        idx = n_tot_vregs - 1 - i
        v = tot_ref[pl.ds(idx * 16, 16)]
        tot = jnp.sum(v)
        newrun = run + tot
        hit = (newrun >= target) & (run < target)
        gi = jnp.where(hit, idx, gi)
        run_before = jnp.where(hit, run, run_before)
        return newrun, gi, run_before

    _, gi, run_g = lax.fori_loop(
        0, n_tot_vregs, body_a, (jnp.int32(0), jnp.int32(0), jnp.int32(0)))
    vt = tot_ref[pl.ds(gi * 16, 16)]
    j1, above1, _ = _stage_b(vt, run_g, target)
    vi = gi * 16 + j1
    vh = h_ref[pl.ds(vi * 16, 16)]
    j2, above2, cnt = _stage_b(vh, above1, target)
    return vi * 16 + j2, above2, cnt


def _totals(h_ref, tot_ref, n_hist_vregs):
    lane = jnp.arange(16, dtype=jnp.int32)
    last = lane == 15

    @plsc.parallel_loop(0, n_hist_vregs, unroll=8)
    def _(i):
        s = plsc.cumsum(h_ref[pl.ds(i * 16, 16)])
        plsc.store_scatter(tot_ref, [lane * 0 + i], s, mask=last)


def _do_row(xv, h1, h2, h3, tot):
    zeros16 = jnp.zeros((16,), jnp.int32)
    ones16 = jnp.ones((16,), jnp.int32)

    @plsc.parallel_loop(0, H1 // 16, unroll=8)
    def _(i):
        h1[pl.ds(i * 16, 16)] = zeros16

    @plsc.parallel_loop(0, H2 // 16, unroll=8)
    def _(i):
        h2[pl.ds(i * 16, 16)] = zeros16
        h3[pl.ds(i * 16, 16)] = zeros16

    @plsc.parallel_loop(0, NV, unroll=8)
    def _(i):
        bits = _bits_of(xv[pl.ds(i * 16, 16)])
        plsc.addupdate_scatter(h1, [bits >> 20], ones16)

    _totals(h1, tot, H1 // 16)
    b1, above1, _ = _find_bucket(h1, tot, H1 // 256, K)
    r2 = K - above1

    @plsc.parallel_loop(0, NV, unroll=8)
    def _(i):
        bits = _bits_of(xv[pl.ds(i * 16, 16)])
        plsc.addupdate_scatter(h2, [(bits >> 10) & 0x3FF], ones16,
                               mask=(bits >> 20) == b1)

    _totals(h2, tot, H2 // 16)
    b2, above2, _ = _find_bucket(h2, tot, H2 // 256, r2)
    r3 = r2 - above2
    hi21 = (b1 << 10) | b2

    @plsc.parallel_loop(0, NV, unroll=8)
    def _(i):
        bits = _bits_of(xv[pl.ds(i * 16, 16)])
        plsc.addupdate_scatter(h3, [bits & 0x3FF], ones16,
                               mask=(bits >> 10) == hi21)

    _totals(h3, tot, H3 // 16)
    b3, above3, meq = _find_bucket(h3, tot, H3 // 256, r3)
    t = (hi21 << 10) | b3
    r = r3 - above3

    def simple(_):
        @plsc.parallel_loop(0, NV, unroll=8)
        def _(i):
            v = xv[pl.ds(i * 16, 16)]
            keep = _bits_of(v) >= t
            xv[pl.ds(i * 16, 16)] = jnp.where(keep, v, 0.0)
        return 0

    def careful(_):
        def body(i, run):
            v = xv[pl.ds(i * 16, 16)]
            bits = _bits_of(v)
            eq = bits == t
            eqi = eq.astype(jnp.int32)
            excl = plsc.cumsum(eqi) - eqi
            keep = (bits > t) | (eq & ((excl + run) < r))
            xv[pl.ds(i * 16, 16)] = jnp.where(keep, v, 0.0)
            return run + jnp.sum(eqi)
        lax.fori_loop(0, NV, body, jnp.int32(0))
        return 0

    lax.cond(r == meq, simple, careful, 0)


def _sc_topk(x_hbm, out_hbm, xv0, xv1, h1, h2, h3, tot,
             sin0, sin1, sout0, sout1):
    wid = lax.axis_index("s") * 2 + lax.axis_index("c")
    base = wid * ROWS_PER_W
    bufs = [(xv0, sin0, sout0), (xv1, sin1, sout1)]
    pltpu.async_copy(x_hbm.at[base], xv0, sin0)
    for j in range(ROWS_PER_W):
        xv, sin, sout = bufs[j % 2]
        if j + 1 < ROWS_PER_W:
            nxv, nsin, nsout = bufs[(j + 1) % 2]
            if j >= 1:
                pltpu.make_async_copy(
                    nxv, out_hbm.at[base + j - 1], nsout).wait()
            pltpu.async_copy(x_hbm.at[base + j + 1], nxv, nsin)
        pltpu.make_async_copy(x_hbm.at[base + j], xv, sin).wait()
        _do_row(xv, h1, h2, h3, tot)
        pltpu.async_copy(xv, out_hbm.at[base + j], sout)
    pltpu.make_async_copy(
        xv0, out_hbm.at[base + ROWS_PER_W - 2], sout0).wait()
    pltpu.make_async_copy(
        xv1, out_hbm.at[base + ROWS_PER_W - 1], sout1).wait()


@jax.jit
def kernel(x):
    f = pl.kernel(
        _sc_topk,
        out_type=jax.ShapeDtypeStruct((ROWS, N), jnp.float32),
        mesh=plsc.VectorSubcoreMesh(core_axis_name="c", subcore_axis_name="s",
                                    num_cores=2, num_subcores=16),
        scratch_types=[
            pltpu.VMEM((N,), jnp.float32),
            pltpu.VMEM((N,), jnp.float32),
            pltpu.VMEM((H1,), jnp.int32),
            pltpu.VMEM((H2,), jnp.int32),
            pltpu.VMEM((H3,), jnp.int32),
            pltpu.VMEM((H1 // 16,), jnp.int32),
            pltpu.SemaphoreType.DMA,
            pltpu.SemaphoreType.DMA,
            pltpu.SemaphoreType.DMA,
            pltpu.SemaphoreType.DMA,
        ],
        compiler_params=pltpu.CompilerParams(needs_layout_passes=False),
    )
    return f(x)
